# baseline (speedup 1.0000x reference)
import jax
import jax.numpy as jnp
from jax.experimental import pallas as pl
from jax.experimental.pallas import tpu as pltpu

_WEIGHT = 2e-9
_SIGMA_RGB = 0.15
_SIGMA_XY = 100.0
_SCALE = 0.5
_OH, _OW = 64, 64
_P = _OH * _OW
_PF = 128 * 128
_KP = 24
_TI = 512
_LOG2E = 1.4426950408889634
_RT = _LOG2E ** 0.5


def _crf_image(img2_ref, seg2_ref, out_ref, ds_ref, sg_ref):
    l_ix = jax.lax.broadcasted_iota(jnp.int32, (512, 128), 1)
    s_ix = jax.lax.broadcasted_iota(jnp.int32, (512, 128), 0)
    base = 256 * (l_ix // 64) + 2 * (l_ix % 64)
    d = s_ix - base
    g_sel = jnp.where(d == 0, 1.0, 0.0).astype(jnp.bfloat16)
    hit = (d == 0) | (d == 1) | (d == 128) | (d == 129)
    g_avg = jnp.where(hit, 0.25, 0.0).astype(jnp.bfloat16)

    img2 = img2_ref[0]
    seg2 = seg2_ref[0]
    for t in range(_P // 128):
        c = slice(512 * t, 512 * (t + 1))
        o = slice(128 * t, 128 * (t + 1))
        ds_ref[:, o] = jax.lax.dot_general(
            img2[:, c], g_sel, (((1,), (0,)), ((), ())),
            preferred_element_type=jnp.float32)
        sg_ref[:, o] = jax.lax.dot_general(
            seg2[:, c], g_avg, (((1,), (0,)), ((), ())),
            preferred_element_type=jnp.float32)

    ds = ds_ref[:, :]
    img_ds = ds[0:3] + ds[3:6]
    roi_ds = ds[6:7] + ds[7:8]
    sgf = sg_ref[:, :]
    seg_ds = (sgf[0:21] + sgf[21:42]) * roi_ds
    seg24 = jnp.concatenate(
        [seg_ds, jnp.zeros((_KP - 21, _P), jnp.float32)], axis=0)
    seg_bb = seg24.astype(jnp.bfloat16)

    sxy = _SIGMA_XY * _SCALE
    p_ix = jax.lax.broadcasted_iota(jnp.int32, (1, _P), 1)
    px = (p_ix % _OW).astype(jnp.float32) * (_RT / sxy)
    py = (p_ix // _OW).astype(jnp.float32) * (_RT / sxy)
    feat5 = jnp.concatenate(
        [px, py, img_ds * (_RT / _SIGMA_RGB)], axis=0)
    m = -0.5 * jnp.sum(feat5 * feat5, axis=0, keepdims=True)
    one = jnp.ones((1, _P), jnp.float32)
    zer5 = jnp.zeros((5, _P), jnp.float32)
    base_l = jnp.concatenate([feat5, feat5, feat5, m, m, one, one, zer5], 0)
    base_r = jnp.concatenate([feat5, feat5, feat5, one, one, m, m, zer5], 0)
    r24 = jax.lax.broadcasted_iota(jnp.int32, (_KP, _P), 0)
    lo_l = ((r24 >= 5) & (r24 < 10)) | (r24 == 16)
    lo_r = ((r24 >= 10) & (r24 < 15)) | (r24 == 18)

    def tobf(b, mask):
        hi = b.astype(jnp.bfloat16)
        lo = (b - hi.astype(jnp.float32)).astype(jnp.bfloat16)
        return jnp.where(mask, lo, hi)

    feat_l = tobf(base_l, lo_l)
    feat_r = tobf(base_r, lo_r)

    acc = jnp.zeros((_KP, _KP), jnp.float32)
    for t in range(_P // _TI):
        o = slice(_TI * t, _TI * (t + 1))
        arg = jax.lax.dot_general(
            feat_l[:, o], feat_r, (((0,), (0,)), ((), ())),
            preferred_element_type=jnp.float32)
        wk = jnp.exp2(jnp.minimum(arg, 0.0)).astype(jnp.bfloat16)
        filt = jax.lax.dot_general(
            wk, seg_bb, (((1,), (1,)), ((), ())),
            preferred_element_type=jnp.float32)
        acc = acc + jax.lax.dot_general(
            seg24[:, o], filt, (((1,), (0,)), ((), ())),
            preferred_element_type=jnp.float32)
    r_ix = jax.lax.broadcasted_iota(jnp.int32, (_KP, _KP), 0)
    c_ix = jax.lax.broadcasted_iota(jnp.int32, (_KP, _KP), 1)
    out_ref[0, 0, :] = jnp.sum(jnp.where(r_ix == c_ix, acc, 0.0), axis=0)


def _split_bf16(x):
    hi = x.astype(jnp.bfloat16)
    lo = (x - hi.astype(jnp.float32)).astype(jnp.bfloat16)
    return hi, lo


def kernel(images, segmentations, ROIs):
    n_img = images.shape[0]
    k_cls = segmentations.shape[1]

    ihi, ilo = _split_bf16(images)
    rhi, rlo = _split_bf16(ROIs[:, None])
    img2 = jnp.concatenate([ihi, ilo, rhi, rlo], axis=1).reshape(
        n_img, 8, _PF)
    shi, slo = _split_bf16(segmentations)
    zp = jnp.zeros((n_img, 48 - 2 * k_cls, 128, 128), jnp.bfloat16)
    seg2 = jnp.concatenate([shi, slo, zp], axis=1).reshape(n_img, 48, _PF)

    partials = pl.pallas_call(
        _crf_image,
        grid=(n_img,),
        in_specs=[
            pl.BlockSpec((1, 8, _PF), lambda p: (p, 0, 0)),
            pl.BlockSpec((1, 48, _PF), lambda p: (p, 0, 0)),
        ],
        out_specs=pl.BlockSpec((1, 1, _KP), lambda p: (p, 0, 0)),
        out_shape=jax.ShapeDtypeStruct((n_img, 1, _KP), jnp.float32),
        scratch_shapes=[
            pltpu.VMEM((8, _P), jnp.float32),
            pltpu.VMEM((48, _P), jnp.float32),
        ],
        compiler_params=pltpu.CompilerParams(
            dimension_semantics=("arbitrary",),
            vmem_limit_bytes=100 * 1024 * 1024,
        ),
    )(img2, seg2)

    return (-_WEIGHT / n_img) * jnp.sum(partials)


# speedup vs baseline: 1.3931x; 1.3931x over previous
import jax
import jax.numpy as jnp
from jax.experimental import pallas as pl
from jax.experimental.pallas import tpu as pltpu

_WEIGHT = 2e-9
_SIGMA_RGB = 0.15
_SIGMA_XY = 100.0
_SCALE = 0.5
_OH, _OW = 64, 64
_P = _OH * _OW
_PF = 128 * 128
_KP = 24
_TI = 512
_LOG2E = 1.4426950408889634
_RT = _LOG2E ** 0.5


def _crf_image(img2_ref, seg2_ref, out_ref, ds_ref, sg_ref):
    l_ix = jax.lax.broadcasted_iota(jnp.int32, (512, 128), 1)
    s_ix = jax.lax.broadcasted_iota(jnp.int32, (512, 128), 0)
    base = 256 * (l_ix // 64) + 2 * (l_ix % 64)
    d = s_ix - base
    g_sel = jnp.where(d == 0, 1.0, 0.0).astype(jnp.bfloat16)
    hit = (d == 0) | (d == 1) | (d == 128) | (d == 129)
    g_avg = jnp.where(hit, 0.25, 0.0).astype(jnp.bfloat16)

    img2 = img2_ref[0]
    seg2 = seg2_ref[0]
    for t in range(_P // 128):
        c = slice(512 * t, 512 * (t + 1))
        o = slice(128 * t, 128 * (t + 1))
        ds_ref[:, o] = jax.lax.dot_general(
            img2[:, c], g_sel, (((1,), (0,)), ((), ())),
            preferred_element_type=jnp.float32)
        sg_ref[:, o] = jax.lax.dot_general(
            seg2[:, c], g_avg, (((1,), (0,)), ((), ())),
            preferred_element_type=jnp.float32)

    ds = ds_ref[:, :]
    img_ds = ds[0:3] + ds[3:6]
    roi_ds = ds[6:7] + ds[7:8]
    sgf = sg_ref[:, :]
    seg_ds = (sgf[0:21] + sgf[21:42]) * roi_ds
    seg24 = jnp.concatenate(
        [seg_ds, jnp.zeros((_KP - 21, _P), jnp.float32)], axis=0)
    seg_bb = seg24.astype(jnp.bfloat16)

    sxy = _SIGMA_XY * _SCALE
    p_ix = jax.lax.broadcasted_iota(jnp.int32, (1, _P), 1)
    px = (p_ix % _OW).astype(jnp.float32) * (_RT / sxy)
    py = (p_ix // _OW).astype(jnp.float32) * (_RT / sxy)
    feat5 = jnp.concatenate(
        [px, py, img_ds * (_RT / _SIGMA_RGB)], axis=0)
    m = -0.5 * jnp.sum(feat5 * feat5, axis=0, keepdims=True)
    one = jnp.ones((1, _P), jnp.float32)
    zer5 = jnp.zeros((5, _P), jnp.float32)
    base_l = jnp.concatenate([feat5, feat5, feat5, m, m, one, one, zer5], 0)
    base_r = jnp.concatenate([feat5, feat5, feat5, one, one, m, m, zer5], 0)
    r24 = jax.lax.broadcasted_iota(jnp.int32, (_KP, _P), 0)
    lo_l = ((r24 >= 5) & (r24 < 10)) | (r24 == 16)
    lo_r = ((r24 >= 10) & (r24 < 15)) | (r24 == 18)

    def tobf(b, mask):
        hi = b.astype(jnp.bfloat16)
        lo = (b - hi.astype(jnp.float32)).astype(jnp.bfloat16)
        return jnp.where(mask, lo, hi)

    feat_l = tobf(base_l, lo_l)
    feat_r = tobf(base_r, lo_r)

    acc = jnp.zeros((_KP, _KP), jnp.float32)
    accd = jnp.zeros((_KP, _KP), jnp.float32)
    for t in range(_P // _TI):
        o = slice(_TI * t, _TI * (t + 1))
        rest = slice(_TI * t, _P)
        arg = jax.lax.dot_general(
            feat_l[:, o], feat_r[:, rest], (((0,), (0,)), ((), ())),
            preferred_element_type=jnp.float32)
        wk = jnp.exp2(jnp.minimum(arg, 0.0)).astype(jnp.bfloat16)
        filt = jax.lax.dot_general(
            wk, seg_bb[:, rest], (((1,), (1,)), ((), ())),
            preferred_element_type=jnp.float32)
        acc = acc + jax.lax.dot_general(
            seg24[:, o], filt, (((1,), (0,)), ((), ())),
            preferred_element_type=jnp.float32)
        filt_d = jax.lax.dot_general(
            wk[:, :_TI], seg_bb[:, o], (((1,), (1,)), ((), ())),
            preferred_element_type=jnp.float32)
        accd = accd + jax.lax.dot_general(
            seg24[:, o], filt_d, (((1,), (0,)), ((), ())),
            preferred_element_type=jnp.float32)
    acc = 2.0 * acc - accd
    r_ix = jax.lax.broadcasted_iota(jnp.int32, (_KP, _KP), 0)
    c_ix = jax.lax.broadcasted_iota(jnp.int32, (_KP, _KP), 1)
    out_ref[0, 0, :] = jnp.sum(jnp.where(r_ix == c_ix, acc, 0.0), axis=0)


def _split_bf16(x):
    hi = x.astype(jnp.bfloat16)
    lo = (x - hi.astype(jnp.float32)).astype(jnp.bfloat16)
    return hi, lo


def kernel(images, segmentations, ROIs):
    n_img = images.shape[0]
    k_cls = segmentations.shape[1]

    ihi, ilo = _split_bf16(images)
    rhi, rlo = _split_bf16(ROIs[:, None])
    img2 = jnp.concatenate([ihi, ilo, rhi, rlo], axis=1).reshape(
        n_img, 8, _PF)
    shi, slo = _split_bf16(segmentations)
    zp = jnp.zeros((n_img, 48 - 2 * k_cls, 128, 128), jnp.bfloat16)
    seg2 = jnp.concatenate([shi, slo, zp], axis=1).reshape(n_img, 48, _PF)

    partials = pl.pallas_call(
        _crf_image,
        grid=(n_img,),
        in_specs=[
            pl.BlockSpec((1, 8, _PF), lambda p: (p, 0, 0)),
            pl.BlockSpec((1, 48, _PF), lambda p: (p, 0, 0)),
        ],
        out_specs=pl.BlockSpec((1, 1, _KP), lambda p: (p, 0, 0)),
        out_shape=jax.ShapeDtypeStruct((n_img, 1, _KP), jnp.float32),
        scratch_shapes=[
            pltpu.VMEM((8, _P), jnp.float32),
            pltpu.VMEM((48, _P), jnp.float32),
        ],
        compiler_params=pltpu.CompilerParams(
            dimension_semantics=("arbitrary",),
            vmem_limit_bytes=100 * 1024 * 1024,
        ),
    )(img2, seg2)

    return (-_WEIGHT / n_img) * jnp.sum(partials)


# speedup vs baseline: 1.7552x; 1.2600x over previous
import jax
import jax.numpy as jnp
from jax.experimental import pallas as pl
from jax.experimental.pallas import tpu as pltpu

_WEIGHT = 2e-9
_SIGMA_RGB = 0.15
_SIGMA_XY = 100.0
_SCALE = 0.5
_OH, _OW = 64, 64
_P = _OH * _OW
_PF = 128 * 128
_KP = 24
_TI = 512
_LOG2E = 1.4426950408889634
_RT = _LOG2E ** 0.5


def _crf_image(img3_ref, seg3_ref, roi3_ref, out_ref, ds_ref, sg_ref):
    l_ix = jax.lax.broadcasted_iota(jnp.int32, (512, 128), 1)
    s_ix = jax.lax.broadcasted_iota(jnp.int32, (512, 128), 0)
    base = 256 * (l_ix // 64) + 2 * (l_ix % 64)
    d = s_ix - base
    g_sel = jnp.where(d == 0, 1.0, 0.0).astype(jnp.bfloat16)
    hit = (d == 0) | (d == 1) | (d == 128) | (d == 129)
    g_avg = jnp.where(hit, 0.25, 0.0).astype(jnp.bfloat16)

    def split(x):
        hi = x.astype(jnp.bfloat16)
        lo = (x - hi.astype(jnp.float32)).astype(jnp.bfloat16)
        return hi, lo

    ihi, ilo = split(img3_ref[0])
    rhi, rlo = split(roi3_ref[0])
    img2 = jnp.concatenate([ihi, ilo, rhi, rlo], axis=0)
    shi, slo = split(seg3_ref[0])
    seg2 = jnp.concatenate([shi, slo], axis=0)
    for t in range(_P // 128):
        c = slice(512 * t, 512 * (t + 1))
        o = slice(128 * t, 128 * (t + 1))
        ds_ref[:, o] = jax.lax.dot_general(
            img2[:, c], g_sel, (((1,), (0,)), ((), ())),
            preferred_element_type=jnp.float32)
        sg_ref[:, o] = jax.lax.dot_general(
            seg2[:, c], g_avg, (((1,), (0,)), ((), ())),
            preferred_element_type=jnp.float32)

    ds = ds_ref[:, :]
    img_ds = ds[0:3] + ds[3:6]
    roi_ds = ds[6:7] + ds[7:8]
    sgf = sg_ref[:, :]
    seg_ds = (sgf[0:21] + sgf[21:42]) * roi_ds
    seg24 = jnp.concatenate(
        [seg_ds, jnp.zeros((_KP - 21, _P), jnp.float32)], axis=0)
    seg_bb = seg24.astype(jnp.bfloat16)

    sxy = _SIGMA_XY * _SCALE
    p_ix = jax.lax.broadcasted_iota(jnp.int32, (1, _P), 1)
    px = (p_ix % _OW).astype(jnp.float32) * (_RT / sxy)
    py = (p_ix // _OW).astype(jnp.float32) * (_RT / sxy)
    feat5 = jnp.concatenate(
        [px, py, img_ds * (_RT / _SIGMA_RGB)], axis=0)
    m = -0.5 * jnp.sum(feat5 * feat5, axis=0, keepdims=True)
    one = jnp.ones((1, _P), jnp.float32)
    zer5 = jnp.zeros((5, _P), jnp.float32)
    base_l = jnp.concatenate([feat5, feat5, feat5, m, m, one, one, zer5], 0)
    base_r = jnp.concatenate([feat5, feat5, feat5, one, one, m, m, zer5], 0)
    r24 = jax.lax.broadcasted_iota(jnp.int32, (_KP, _P), 0)
    lo_l = ((r24 >= 5) & (r24 < 10)) | (r24 == 16)
    lo_r = ((r24 >= 10) & (r24 < 15)) | (r24 == 18)

    def tobf(b, mask):
        hi = b.astype(jnp.bfloat16)
        lo = (b - hi.astype(jnp.float32)).astype(jnp.bfloat16)
        return jnp.where(mask, lo, hi)

    feat_l = tobf(base_l, lo_l)
    feat_r = tobf(base_r, lo_r)

    acc = jnp.zeros((_KP, _KP), jnp.float32)
    accd = jnp.zeros((_KP, _KP), jnp.float32)
    for t in range(_P // _TI):
        o = slice(_TI * t, _TI * (t + 1))
        rest = slice(_TI * t, _P)
        arg = jax.lax.dot_general(
            feat_l[:, o], feat_r[:, rest], (((0,), (0,)), ((), ())),
            preferred_element_type=jnp.float32)
        wk = jnp.exp2(jnp.minimum(arg, 0.0)).astype(jnp.bfloat16)
        filt = jax.lax.dot_general(
            wk, seg_bb[:, rest], (((1,), (1,)), ((), ())),
            preferred_element_type=jnp.float32)
        acc = acc + jax.lax.dot_general(
            seg24[:, o], filt, (((1,), (0,)), ((), ())),
            preferred_element_type=jnp.float32)
        filt_d = jax.lax.dot_general(
            wk[:, :_TI], seg_bb[:, o], (((1,), (1,)), ((), ())),
            preferred_element_type=jnp.float32)
        accd = accd + jax.lax.dot_general(
            seg24[:, o], filt_d, (((1,), (0,)), ((), ())),
            preferred_element_type=jnp.float32)
    acc = 2.0 * acc - accd
    r_ix = jax.lax.broadcasted_iota(jnp.int32, (_KP, _KP), 0)
    c_ix = jax.lax.broadcasted_iota(jnp.int32, (_KP, _KP), 1)
    out_ref[0, 0, :] = jnp.sum(jnp.where(r_ix == c_ix, acc, 0.0), axis=0)


def kernel(images, segmentations, ROIs):
    n_img = images.shape[0]
    k_cls = segmentations.shape[1]

    img3 = images.reshape(n_img, 3, _PF)
    seg3 = segmentations.reshape(n_img, k_cls, _PF)
    roi3 = ROIs.reshape(n_img, 1, _PF)

    partials = pl.pallas_call(
        _crf_image,
        grid=(n_img,),
        in_specs=[
            pl.BlockSpec((1, 3, _PF), lambda p: (p, 0, 0)),
            pl.BlockSpec((1, k_cls, _PF), lambda p: (p, 0, 0)),
            pl.BlockSpec((1, 1, _PF), lambda p: (p, 0, 0)),
        ],
        out_specs=pl.BlockSpec((1, 1, _KP), lambda p: (p, 0, 0)),
        out_shape=jax.ShapeDtypeStruct((n_img, 1, _KP), jnp.float32),
        scratch_shapes=[
            pltpu.VMEM((8, _P), jnp.float32),
            pltpu.VMEM((42, _P), jnp.float32),
        ],
        compiler_params=pltpu.CompilerParams(
            dimension_semantics=("arbitrary",),
            vmem_limit_bytes=100 * 1024 * 1024,
        ),
    )(img3, seg3, roi3)

    return (-_WEIGHT / n_img) * jnp.sum(partials)
